# baseline (speedup 1.0000x reference)
import functools

import jax
import jax.numpy as jnp
from jax import lax
from jax.experimental import pallas as pl
from jax.experimental.pallas import tpu as pltpu
from jax.experimental.pallas import tpu_sc as plsc

NUM_BLOCKS = 16
BLOCK_SIZE = 256
BLOCK_RANK = 256
TOP_K = 8

N_ROWS = 8192
ROW_TILE = 2048

SC_CORES = 2
SC_SUBCORES = 16
SC_WORKERS = SC_CORES * SC_SUBCORES
ROWS_PER_W = N_ROWS // SC_WORKERS


def _routing_sc_kernel(idx_hbm, score_hbm, w_hbm, idx_v, score_v, w_v):
    wid = lax.axis_index("s") * SC_CORES + lax.axis_index("c")
    base = wid * ROWS_PER_W
    pltpu.sync_copy(idx_hbm.at[pl.ds(base, ROWS_PER_W)], idx_v)
    pltpu.sync_copy(score_hbm.at[pl.ds(base, ROWS_PER_W)], score_v)
    riota = lax.iota(jnp.int32, 16)
    zero16 = jnp.zeros((16,), jnp.float32)
    for r in range(ROWS_PER_W):
        w_v[r, :] = zero16
    for c in range(ROWS_PER_W // 16):
        rows16 = riota + (c * 16)
        cols = [jnp.full((16,), k, jnp.int32) for k in range(TOP_K)]
        s = [plsc.load_gather(score_v, [rows16, cols[k]]) for k in range(TOP_K)]
        m = s[0]
        for k in range(1, TOP_K):
            m = jnp.maximum(m, s[k])
        ex = [jnp.exp(sk - m) for sk in s]
        tot = ex[0]
        for k in range(1, TOP_K):
            tot = tot + ex[k]
        inv = 1.0 / tot
        for k in range(TOP_K):
            ik = plsc.load_gather(idx_v, [rows16, cols[k]])
            plsc.addupdate_scatter(w_v, [rows16, ik], ex[k] * inv)
    plsc.subcore_barrier()
    pltpu.sync_copy(w_v, w_hbm.at[pl.ds(base, ROWS_PER_W)])


_routing_sc = pl.kernel(
    _routing_sc_kernel,
    out_type=jax.ShapeDtypeStruct((N_ROWS, NUM_BLOCKS), jnp.float32),
    mesh=plsc.VectorSubcoreMesh(
        core_axis_name="c", subcore_axis_name="s",
        num_cores=SC_CORES, num_subcores=SC_SUBCORES),
    scratch_types=[
        pltpu.VMEM((ROWS_PER_W, TOP_K), jnp.int32),
        pltpu.VMEM((ROWS_PER_W, TOP_K), jnp.float32),
        pltpu.VMEM((ROWS_PER_W, NUM_BLOCKS), jnp.float32),
    ],
    compiler_params=pltpu.CompilerParams(needs_layout_passes=False),
)


def _adapter_kernel(w_ref, x_ref, dw_ref, db_ref, uw_ref, ub_ref, out_ref):
    e = pl.program_id(0)
    lane = jax.lax.broadcasted_iota(jnp.int32, (1, NUM_BLOCKS), 1)
    w = jnp.sum(jnp.where(lane == e, w_ref[...], 0.0), axis=1, keepdims=True)

    x = x_ref[...].astype(jnp.bfloat16)
    dw = dw_ref[0].astype(jnp.bfloat16)
    uw = uw_ref[0].astype(jnp.bfloat16)
    h = jnp.dot(x, dw,
                preferred_element_type=jnp.float32).astype(jnp.bfloat16)
    h = h + db_ref[0]
    hh = h * jnp.bfloat16(0.5)
    act = hh + hh * jnp.tanh(hh)
    out = jnp.dot(act, uw, preferred_element_type=jnp.float32) + ub_ref[0]
    out_ref[...] = out * w


@jax.jit
def kernel(hidden_states, active_idx, active_score, down_w, down_b, up_w, up_b):
    batch, seq_len, hidden = hidden_states.shape
    n_rows = batch * seq_len
    x2d = hidden_states.reshape(n_rows, hidden)
    n_tiles = n_rows // ROW_TILE

    w_mat = _routing_sc(active_idx, active_score)

    grid = (NUM_BLOCKS, n_tiles)
    out = pl.pallas_call(
        _adapter_kernel,
        grid=grid,
        in_specs=[
            pl.BlockSpec((ROW_TILE, NUM_BLOCKS), lambda e, t: (t, 0)),
            pl.BlockSpec((ROW_TILE, BLOCK_SIZE), lambda e, t: (t, e)),
            pl.BlockSpec((1, BLOCK_SIZE, BLOCK_RANK), lambda e, t: (e, 0, 0)),
            pl.BlockSpec((1, 1, BLOCK_RANK), lambda e, t: (e, 0, 0)),
            pl.BlockSpec((1, BLOCK_RANK, BLOCK_SIZE), lambda e, t: (e, 0, 0)),
            pl.BlockSpec((1, 1, BLOCK_SIZE), lambda e, t: (e, 0, 0)),
        ],
        out_specs=pl.BlockSpec((ROW_TILE, BLOCK_SIZE), lambda e, t: (t, e)),
        out_shape=jax.ShapeDtypeStruct((n_rows, hidden), jnp.float32),
    )(w_mat, x2d, down_w,
      down_b.reshape(NUM_BLOCKS, 1, BLOCK_RANK).astype(jnp.bfloat16), up_w,
      up_b.reshape(NUM_BLOCKS, 1, BLOCK_SIZE))
    return out.reshape(batch, seq_len, hidden)
